# Initial kernel scaffold; baseline (speedup 1.0000x reference)
import jax
import jax.numpy as jnp
from jax.experimental import pallas as pl


def kernel(inputs):
    raise NotImplementedError("write your pallas kernel here")


# speedup vs baseline: 6.6908x; 6.6908x over previous
import jax
import jax.numpy as jnp
from jax.experimental import pallas as pl

N_FILTER = 256
KEEP = 48 * N_FILTER
BLOCK_ROWS = 128


def _copy_kernel(in_ref, out_ref):
    out_ref[...] = in_ref[...]


def kernel(inputs):
    rows = inputs.shape[0]
    grid = (rows // BLOCK_ROWS,)
    return pl.pallas_call(
        _copy_kernel,
        grid=grid,
        in_specs=[
            pl.BlockSpec((BLOCK_ROWS, KEEP), lambda i: (i, 0)),
        ],
        out_specs=pl.BlockSpec((BLOCK_ROWS, KEEP), lambda i: (i, 0)),
        out_shape=jax.ShapeDtypeStruct((rows, KEEP), inputs.dtype),
    )(inputs)


# speedup vs baseline: 6.6954x; 1.0007x over previous
import jax
import jax.numpy as jnp
from jax.experimental import pallas as pl

N_FILTER = 256
KEEP = 48 * N_FILTER
BLOCK_ROWS = 256


def _copy_kernel(in_ref, out_ref):
    out_ref[...] = in_ref[...]


def kernel(inputs):
    rows = inputs.shape[0]
    grid = (rows // BLOCK_ROWS,)
    return pl.pallas_call(
        _copy_kernel,
        grid=grid,
        in_specs=[
            pl.BlockSpec((BLOCK_ROWS, KEEP), lambda i: (i, 0)),
        ],
        out_specs=pl.BlockSpec((BLOCK_ROWS, KEEP), lambda i: (i, 0)),
        out_shape=jax.ShapeDtypeStruct((rows, KEEP), inputs.dtype),
    )(inputs)
